# Initial kernel scaffold; baseline (speedup 1.0000x reference)
import jax
import jax.numpy as jnp
from jax.experimental import pallas as pl


def kernel(inputs, codebook):
    raise NotImplementedError("write your pallas kernel here")


# speedup vs baseline: 2.3443x; 2.3443x over previous
import jax
import jax.numpy as jnp
from jax.experimental import pallas as pl
from jax.experimental.pallas import tpu as pltpu

_NUM_EMB = 128
_DIM = 64
_N_ROWS = 65536
_BLK = 2048
_NSTEPS = _N_ROWS // _BLK
_INV_ND = 1.0 / (_N_ROWS * _DIM)


def _vq_body(x_ref, cb_ref, idx_ref, q_ref, loss_ref, perp_ref,
             counts_acc, loss_acc):
    i = pl.program_id(0)
    x = x_ref[...]
    cb = cb_ref[...]
    x2 = jnp.sum(x * x, axis=1, keepdims=True)
    c2 = jnp.sum(cb * cb, axis=1)
    mm = jax.lax.dot_general(x, cb, (((1,), (1,)), ((), ())),
                             preferred_element_type=jnp.float32)
    d = (x2 + c2) - 2.0 * mm
    min_d = jnp.min(d, axis=1, keepdims=True)
    iota = jax.lax.broadcasted_iota(jnp.int32, (_BLK, _NUM_EMB), 1)
    idx = jnp.min(jnp.where(d == min_d, iota, _NUM_EMB), axis=1)
    onehot = (iota == idx[:, None]).astype(jnp.float32)
    q = jax.lax.dot_general(onehot, cb, (((1,), (0,)), ((), ())),
                            preferred_element_type=jnp.float32)
    idx_ref[...] = idx.reshape(1, 1, _BLK)
    q_ref[...] = q

    @pl.when(i == 0)
    def _init():
        counts_acc[...] = jnp.zeros_like(counts_acc)
        loss_acc[...] = jnp.zeros_like(loss_acc)

    counts_acc[...] += jnp.sum(onehot, axis=0, keepdims=True)
    loss_acc[...] += jnp.sum(min_d, axis=(0, 1), keepdims=True)

    @pl.when(i == _NSTEPS - 1)
    def _finalize():
        loss_ref[...] = 1.25 * _INV_ND * loss_acc[...]
        avg = counts_acc[...] * (1.0 / _N_ROWS)
        ent = jnp.sum(avg * jnp.log(avg + 1e-10), axis=1, keepdims=True)
        perp_ref[...] = jnp.exp(-ent)


def kernel(inputs, codebook):
    flat = inputs.reshape(_N_ROWS, _DIM)
    idx3, q, loss, perp = pl.pallas_call(
        _vq_body,
        grid=(_NSTEPS,),
        in_specs=[
            pl.BlockSpec((_BLK, _DIM), lambda i: (i, 0)),
            pl.BlockSpec((_NUM_EMB, _DIM), lambda i: (0, 0)),
        ],
        out_specs=[
            pl.BlockSpec((1, 1, _BLK), lambda i: (i, 0, 0)),
            pl.BlockSpec((_BLK, _DIM), lambda i: (i, 0)),
            pl.BlockSpec((1, 1), lambda i: (0, 0)),
            pl.BlockSpec((1, 1), lambda i: (0, 0)),
        ],
        out_shape=[
            jax.ShapeDtypeStruct((_NSTEPS, 1, _BLK), jnp.int32),
            jax.ShapeDtypeStruct((_N_ROWS, _DIM), jnp.float32),
            jax.ShapeDtypeStruct((1, 1), jnp.float32),
            jax.ShapeDtypeStruct((1, 1), jnp.float32),
        ],
        scratch_shapes=[
            pltpu.VMEM((1, _NUM_EMB), jnp.float32),
            pltpu.VMEM((1, 1), jnp.float32),
        ],
        compiler_params=pltpu.CompilerParams(
            dimension_semantics=("arbitrary",)),
    )(flat, codebook)
    return (loss[0, 0], q.reshape(inputs.shape), perp[0, 0],
            idx3.reshape(_N_ROWS))


# speedup vs baseline: 2.9521x; 1.2593x over previous
import jax
import jax.numpy as jnp
from jax.experimental import pallas as pl
from jax.experimental.pallas import tpu as pltpu

_NUM_EMB = 128
_DIM = 64
_N_ROWS = 65536
_BLK = 2048
_NSTEPS = _N_ROWS // _BLK
_INV_ND = 1.0 / (_N_ROWS * _DIM)


def _vq_body(x_ref, cb_ref, idx_ref, q_ref, loss_ref, perp_ref,
             counts_acc, loss_acc):
    i = pl.program_id(0)
    x = x_ref[...]
    cb = cb_ref[...]
    x2 = jnp.sum(x * x, axis=1, keepdims=True)
    c2 = jnp.sum(cb * cb, axis=1)
    mm = jax.lax.dot_general(x, cb, (((1,), (1,)), ((), ())),
                             preferred_element_type=jnp.float32)
    d = (x2 + c2) - 2.0 * mm
    min_d = jnp.min(d, axis=1, keepdims=True)
    iota1 = jax.lax.broadcasted_iota(
        jnp.int32, (1, _NUM_EMB), 1).astype(jnp.float32)
    t = jnp.where(d == min_d, 128.0 - iota1, 0.0)
    rmax = jnp.max(t, axis=1, keepdims=True)
    oh = jnp.where(t == rmax, 1.0, 0.0)
    q = jax.lax.dot_general(oh, cb, (((1,), (0,)), ((), ())),
                            preferred_element_type=jnp.float32)
    idxf = jax.lax.dot_general(iota1, oh, (((1,), (1,)), ((), ())),
                               preferred_element_type=jnp.float32)
    idx_ref[...] = idxf.astype(jnp.int32).reshape(_BLK)
    q_ref[...] = q

    @pl.when(i == 0)
    def _init():
        counts_acc[...] = jnp.zeros_like(counts_acc)
        loss_acc[...] = jnp.zeros_like(loss_acc)

    ones1 = jnp.ones((1, _BLK), jnp.float32)
    counts_acc[...] += jax.lax.dot_general(
        ones1, oh, (((1,), (0,)), ((), ())),
        preferred_element_type=jnp.float32)
    loss_acc[...] += jax.lax.dot_general(
        ones1, min_d, (((1,), (0,)), ((), ())),
        preferred_element_type=jnp.float32)

    @pl.when(i == _NSTEPS - 1)
    def _finalize():
        loss_ref[...] = 1.25 * _INV_ND * loss_acc[...]
        avg = counts_acc[...] * (1.0 / _N_ROWS)
        ent = jnp.sum(avg * jnp.log(avg + 1e-10), axis=1, keepdims=True)
        perp_ref[...] = jnp.exp(-ent)


def kernel(inputs, codebook):
    flat = inputs.reshape(_N_ROWS, _DIM)
    idx, q, loss, perp = pl.pallas_call(
        _vq_body,
        grid=(_NSTEPS,),
        in_specs=[
            pl.BlockSpec((_BLK, _DIM), lambda i: (i, 0)),
            pl.BlockSpec((_NUM_EMB, _DIM), lambda i: (0, 0)),
        ],
        out_specs=[
            pl.BlockSpec((_BLK,), lambda i: (i,)),
            pl.BlockSpec((_BLK, _DIM), lambda i: (i, 0)),
            pl.BlockSpec((1, 1), lambda i: (0, 0)),
            pl.BlockSpec((1, 1), lambda i: (0, 0)),
        ],
        out_shape=[
            jax.ShapeDtypeStruct((_N_ROWS,), jnp.int32),
            jax.ShapeDtypeStruct((_N_ROWS, _DIM), jnp.float32),
            jax.ShapeDtypeStruct((1, 1), jnp.float32),
            jax.ShapeDtypeStruct((1, 1), jnp.float32),
        ],
        scratch_shapes=[
            pltpu.VMEM((1, _NUM_EMB), jnp.float32),
            pltpu.VMEM((1, 1), jnp.float32),
        ],
        compiler_params=pltpu.CompilerParams(
            dimension_semantics=("arbitrary",)),
    )(flat, codebook)
    return (loss[0, 0], q.reshape(inputs.shape), perp[0, 0], idx)


# speedup vs baseline: 4.8464x; 1.6417x over previous
import jax
import jax.numpy as jnp
from jax.experimental import pallas as pl
from jax.experimental.pallas import tpu as pltpu

_NUM_EMB = 128
_DIM = 64
_BATCH = 64
_SEQ = 1024
_N_ROWS = _BATCH * _SEQ
_INV_ND = 1.0 / (_N_ROWS * _DIM)


def _vq_body(x_ref, cb_ref, idx_ref, q_ref, loss_ref, perp_ref,
             counts_acc, loss_acc):
    b = pl.program_id(0)
    xt = x_ref[0]
    cb = cb_ref[...]
    x2 = jnp.sum(xt * xt, axis=0, keepdims=True)
    c2 = jnp.sum(cb * cb, axis=1, keepdims=True)
    mm = jax.lax.dot_general(cb, xt, (((1,), (0,)), ((), ())),
                             preferred_element_type=jnp.float32)
    d = (x2 + c2) - 2.0 * mm
    min_d = jnp.min(d, axis=0, keepdims=True)
    rev = 128.0 - jax.lax.broadcasted_iota(
        jnp.int32, (_NUM_EMB, 1), 0).astype(jnp.float32)
    t = jnp.where(d == min_d, rev, 0.0)
    rmax = jnp.max(t, axis=0, keepdims=True)
    oh = jnp.where(t == rmax, 1.0, 0.0)
    qt = jax.lax.dot_general(cb, oh, (((0,), (0,)), ((), ())),
                             preferred_element_type=jnp.float32)
    idx_ref[...] = (128.0 - rmax).astype(jnp.int32).reshape(_SEQ)
    q_ref[0] = qt

    @pl.when(b == 0)
    def _init():
        counts_acc[...] = jnp.zeros_like(counts_acc)
        loss_acc[...] = jnp.zeros_like(loss_acc)

    ones_s = jnp.ones((_SEQ, 1), jnp.float32)
    counts_acc[...] += jax.lax.dot_general(
        oh, ones_s, (((1,), (0,)), ((), ())),
        preferred_element_type=jnp.float32)
    loss_acc[...] += jax.lax.dot_general(
        min_d, ones_s, (((1,), (0,)), ((), ())),
        preferred_element_type=jnp.float32)

    @pl.when(b == pl.num_programs(0) - 1)
    def _finalize():
        loss_ref[...] = 1.25 * _INV_ND * loss_acc[...]
        avg = counts_acc[...] * (1.0 / _N_ROWS)
        ent = jnp.sum(avg * jnp.log(avg + 1e-10), axis=0, keepdims=True)
        perp_ref[...] = jnp.exp(-ent)


def kernel(inputs, codebook):
    xt = jnp.transpose(inputs, (0, 2, 1))
    idx, qt, loss, perp = pl.pallas_call(
        _vq_body,
        grid=(_BATCH,),
        in_specs=[
            pl.BlockSpec((1, _DIM, _SEQ), lambda b: (b, 0, 0)),
            pl.BlockSpec((_NUM_EMB, _DIM), lambda b: (0, 0)),
        ],
        out_specs=[
            pl.BlockSpec((_SEQ,), lambda b: (b,)),
            pl.BlockSpec((1, _DIM, _SEQ), lambda b: (b, 0, 0)),
            pl.BlockSpec((1, 1), lambda b: (0, 0)),
            pl.BlockSpec((1, 1), lambda b: (0, 0)),
        ],
        out_shape=[
            jax.ShapeDtypeStruct((_N_ROWS,), jnp.int32),
            jax.ShapeDtypeStruct((_BATCH, _DIM, _SEQ), jnp.float32),
            jax.ShapeDtypeStruct((1, 1), jnp.float32),
            jax.ShapeDtypeStruct((1, 1), jnp.float32),
        ],
        scratch_shapes=[
            pltpu.VMEM((_NUM_EMB, 1), jnp.float32),
            pltpu.VMEM((1, 1), jnp.float32),
        ],
        compiler_params=pltpu.CompilerParams(
            dimension_semantics=("arbitrary",)),
    )(xt, codebook)
    q = jnp.transpose(qt, (0, 2, 1))
    return (loss[0, 0], q, perp[0, 0], idx)


# speedup vs baseline: 8.3059x; 1.7138x over previous
import jax
import jax.numpy as jnp
from jax.experimental import pallas as pl
from jax.experimental.pallas import tpu as pltpu

_NUM_EMB = 128
_DIM = 64
_BATCH = 64
_SEQ = 1024
_P = 8
_NSTEPS = _BATCH // _P
_N_ROWS = _BATCH * _SEQ
_INV_ND = 1.0 / (_N_ROWS * _DIM)


def _vq_body(x_ref, cb_ref, idx_ref, q_ref, loss_ref, perp_ref,
             counts_acc, loss_acc):
    i = pl.program_id(0)
    cb = cb_ref[...]
    c2 = jnp.sum(cb * cb, axis=1, keepdims=True)
    rev = 128.0 - jax.lax.broadcasted_iota(
        jnp.int32, (_NUM_EMB, 1), 0).astype(jnp.float32)
    ones_s = jnp.ones((_SEQ, 1), jnp.float32)

    @pl.when(i == 0)
    def _init():
        counts_acc[...] = jnp.zeros_like(counts_acc)
        loss_acc[...] = jnp.zeros_like(loss_acc)

    counts_new = counts_acc[...]
    loss_new = loss_acc[...]

    for p in range(_P):
        xt = x_ref[p]
        x2 = jnp.sum(xt * xt, axis=0, keepdims=True)
        mm = jax.lax.dot_general(cb, xt, (((1,), (0,)), ((), ())),
                                 preferred_element_type=jnp.float32)
        d = (x2 + c2) - 2.0 * mm
        min_d = jnp.min(d, axis=0, keepdims=True)
        t = jnp.where(d == min_d, rev, 0.0)
        rmax = jnp.max(t, axis=0, keepdims=True)
        oh = jnp.where(t == rmax, 1.0, 0.0)
        qt = jax.lax.dot_general(cb, oh, (((0,), (0,)), ((), ())),
                                 preferred_element_type=jnp.float32)
        idx_ref[pl.ds(p * _SEQ, _SEQ)] = (
            (128.0 - rmax).astype(jnp.int32).reshape(_SEQ))
        q_ref[p] = qt
        counts_new += jax.lax.dot_general(
            oh, ones_s, (((1,), (0,)), ((), ())),
            preferred_element_type=jnp.float32)
        loss_new += jax.lax.dot_general(
            min_d, ones_s, (((1,), (0,)), ((), ())),
            preferred_element_type=jnp.float32)

    counts_acc[...] = counts_new
    loss_acc[...] = loss_new

    @pl.when(i == _NSTEPS - 1)
    def _finalize():
        loss_ref[...] = 1.25 * _INV_ND * loss_acc[...]
        avg = counts_acc[...] * (1.0 / _N_ROWS)
        ent = jnp.sum(avg * jnp.log(avg + 1e-10), axis=0, keepdims=True)
        perp_ref[...] = jnp.exp(-ent)


def kernel(inputs, codebook):
    xt = jnp.transpose(inputs, (0, 2, 1))
    idx, qt, loss, perp = pl.pallas_call(
        _vq_body,
        grid=(_NSTEPS,),
        in_specs=[
            pl.BlockSpec((_P, _DIM, _SEQ), lambda i: (i, 0, 0)),
            pl.BlockSpec((_NUM_EMB, _DIM), lambda i: (0, 0)),
        ],
        out_specs=[
            pl.BlockSpec((_P * _SEQ,), lambda i: (i,)),
            pl.BlockSpec((_P, _DIM, _SEQ), lambda i: (i, 0, 0)),
            pl.BlockSpec((1, 1), lambda i: (0, 0)),
            pl.BlockSpec((1, 1), lambda i: (0, 0)),
        ],
        out_shape=[
            jax.ShapeDtypeStruct((_N_ROWS,), jnp.int32),
            jax.ShapeDtypeStruct((_BATCH, _DIM, _SEQ), jnp.float32),
            jax.ShapeDtypeStruct((1, 1), jnp.float32),
            jax.ShapeDtypeStruct((1, 1), jnp.float32),
        ],
        scratch_shapes=[
            pltpu.VMEM((_NUM_EMB, 1), jnp.float32),
            pltpu.VMEM((1, 1), jnp.float32),
        ],
        compiler_params=pltpu.CompilerParams(
            dimension_semantics=("arbitrary",)),
    )(xt, codebook)
    q = jnp.transpose(qt, (0, 2, 1))
    return (loss[0, 0], q, perp[0, 0], idx)
